# baseline (speedup 1.0000x reference)
import functools

import jax
import jax.numpy as jnp
from jax import lax
from jax.experimental import pallas as pl
from jax.experimental.pallas import tpu as pltpu
from jax.experimental.pallas import tpu_sc as plsc

B, N = 512, 256
INPUT_DIM, COND_DIM, HIDDEN, LATENT, K = 3, 16, 256, 64, 1024
BM = 128
TCHUNK = 64
NPOS = 256
NVALID = N - 2

SC_NC, SC_NS = 2, 16
SC_NW = SC_NC * SC_NS
B_PER_W = B // SC_NW


def _enc_kernel(x3_ref, w9_ref, cb_b_ref, w1_ref, b1_ref, w2_ref, b2_ref,
                wl_ref, bl_ref, cbt_ref, c_ref, n_ref, wc_ref, wn_ref,
                bd_ref, idx_ref, part_ref):
    run = jnp.full((BM, HIDDEN), -jnp.inf, jnp.float32)
    for i in range(NPOS // TCHUNK):
        xw = x3_ref[:, i * TCHUNK:(i + 1) * TCHUNK, :]
        hp = jnp.dot(xw.reshape(BM * TCHUNK, 9), w9_ref[:],
                     preferred_element_type=jnp.float32)
        hp = hp.reshape(BM, TCHUNK, HIDDEN)
        t0 = i * TCHUNK
        if t0 + TCHUNK > NVALID:
            tloc = lax.broadcasted_iota(jnp.int32, (BM, TCHUNK, HIDDEN), 1)
            hp = jnp.where(t0 + tloc < NVALID, hp, -jnp.inf)
        run = jnp.maximum(run, jnp.max(hp, axis=1))
    h = run + cb_b_ref[:]
    h = jnp.maximum(jnp.dot(h, w1_ref[:], preferred_element_type=jnp.float32)
                    + b1_ref[:], 0.0)
    h = jnp.maximum(jnp.dot(h, w2_ref[:], preferred_element_type=jnp.float32)
                    + b2_ref[:], 0.0)
    z = jnp.dot(h, wl_ref[:], preferred_element_type=jnp.float32) + bl_ref[:]
    dist = cbt_ref[0:1, :] - 2.0 * jnp.dot(z, cbt_ref[1:, :],
                                           preferred_element_type=jnp.float32)
    dmin = jnp.min(dist, axis=1, keepdims=True)
    ks = lax.broadcasted_iota(jnp.int32, (BM, K), 1)
    idx = jnp.min(jnp.where(dist <= dmin, ks, K), axis=1)
    idx_ref[:, 0] = idx
    part_ref[:, :] = (jnp.dot(c_ref[:], wc_ref[:], preferred_element_type=jnp.float32)
                      + jnp.dot(n_ref[:], wn_ref[:], preferred_element_type=jnp.float32)
                      + bd_ref[:])


def _dec_kernel(zq_ref, part_ref, wz_ref, w1_ref, b1_ref, w2_ref, b2_ref,
                wo_ref, bo_ref, out_ref):
    d = jnp.maximum(jnp.dot(zq_ref[:], wz_ref[:],
                            preferred_element_type=jnp.float32) + part_ref[:], 0.0)
    d = jnp.maximum(jnp.dot(d, w1_ref[:], preferred_element_type=jnp.float32)
                    + b1_ref[:], 0.0)
    d = jnp.maximum(jnp.dot(d, w2_ref[:], preferred_element_type=jnp.float32)
                    + b2_ref[:], 0.0)
    out_ref[:, :] = (jnp.dot(d, wo_ref[:], preferred_element_type=jnp.float32)
                     + bo_ref[:])


def _sc_gather(codebook, idx):
    mesh = plsc.VectorSubcoreMesh(core_axis_name="c", subcore_axis_name="s")

    @functools.partial(
        pl.kernel, mesh=mesh,
        out_type=jax.ShapeDtypeStruct((B, LATENT), jnp.float32),
        compiler_params=pltpu.CompilerParams(use_tc_tiling_on_sc=False),
        scratch_types=[
            pltpu.VMEM((B_PER_W,), jnp.int32),
            pltpu.VMEM((B_PER_W, LATENT), jnp.float32),
            pltpu.SemaphoreType.DMA,
        ],
    )
    def gather(table_hbm, idx_hbm, out_hbm, idx_v, rows_v, sem):
        wid = lax.axis_index("s") * SC_NC + lax.axis_index("c")
        base = wid * B_PER_W
        pltpu.sync_copy(idx_hbm.at[pl.ds(base, B_PER_W)], idx_v)
        pltpu.async_copy(table_hbm.at[idx_v], rows_v, sem).wait()
        pltpu.sync_copy(rows_v, out_hbm.at[pl.ds(base, B_PER_W)])

    return gather(codebook, idx)


def kernel(x, c, noise, conv_w, conv_b, enc_h1_w, enc_h1_b, enc_h2_w, enc_h2_b,
           enc_lat_w, enc_lat_b, codebook, dec_in_w, dec_in_b, dec_h1_w, dec_h1_b,
           dec_h2_w, dec_h2_b, dec_out_w, dec_out_b):
    xp = jnp.pad(x, ((0, 0), (0, 2), (0, 0)))
    x3 = jnp.concatenate([xp[:, 0:NPOS, :], xp[:, 1:NPOS + 1, :],
                          xp[:, 2:NPOS + 2, :]], axis=2)
    w9 = jnp.transpose(conv_w, (2, 1, 0)).reshape(9, HIDDEN)
    cbt = jnp.concatenate([jnp.sum(codebook ** 2, axis=1)[None, :],
                           codebook.T], axis=0)
    wz = dec_in_w[:LATENT]
    wc = dec_in_w[LATENT:LATENT + COND_DIM]
    wn = dec_in_w[LATENT + COND_DIM:]
    row = lambda v: v[None, :]

    nb = B // BM
    idx2, part = pl.pallas_call(
        _enc_kernel,
        grid=(nb,),
        in_specs=[
            pl.BlockSpec((BM, NPOS, 9), lambda i: (i, 0, 0)),
            pl.BlockSpec((9, HIDDEN), lambda i: (0, 0)),
            pl.BlockSpec((1, HIDDEN), lambda i: (0, 0)),
            pl.BlockSpec((HIDDEN, HIDDEN), lambda i: (0, 0)),
            pl.BlockSpec((1, HIDDEN), lambda i: (0, 0)),
            pl.BlockSpec((HIDDEN, HIDDEN), lambda i: (0, 0)),
            pl.BlockSpec((1, HIDDEN), lambda i: (0, 0)),
            pl.BlockSpec((HIDDEN, LATENT), lambda i: (0, 0)),
            pl.BlockSpec((1, LATENT), lambda i: (0, 0)),
            pl.BlockSpec((1 + LATENT, K), lambda i: (0, 0)),
            pl.BlockSpec((BM, COND_DIM), lambda i: (i, 0)),
            pl.BlockSpec((BM, INPUT_DIM), lambda i: (i, 0)),
            pl.BlockSpec((COND_DIM, HIDDEN), lambda i: (0, 0)),
            pl.BlockSpec((INPUT_DIM, HIDDEN), lambda i: (0, 0)),
            pl.BlockSpec((1, HIDDEN), lambda i: (0, 0)),
        ],
        out_specs=[
            pl.BlockSpec((BM, 1), lambda i: (i, 0)),
            pl.BlockSpec((BM, HIDDEN), lambda i: (i, 0)),
        ],
        out_shape=[
            jax.ShapeDtypeStruct((B, 1), jnp.int32),
            jax.ShapeDtypeStruct((B, HIDDEN), jnp.float32),
        ],
        compiler_params=pltpu.CompilerParams(
            dimension_semantics=("arbitrary",)),
    )(x3, w9, row(conv_b), enc_h1_w, row(enc_h1_b), enc_h2_w, row(enc_h2_b),
      enc_lat_w, row(enc_lat_b), cbt, c, noise, wc, wn, row(dec_in_b))

    z_q = _sc_gather(codebook, idx2.reshape(B))

    out = pl.pallas_call(
        _dec_kernel,
        out_shape=jax.ShapeDtypeStruct((B, INPUT_DIM), jnp.float32),
    )(z_q, part, wz, dec_h1_w, row(dec_h1_b), dec_h2_w, row(dec_h2_b),
      dec_out_w, row(dec_out_b))
    return out


# speedup vs baseline: 1.5735x; 1.5735x over previous
import functools

import jax
import jax.numpy as jnp
from jax import lax
from jax.experimental import pallas as pl
from jax.experimental.pallas import tpu as pltpu
from jax.experimental.pallas import tpu_sc as plsc

B, N = 512, 256
INPUT_DIM, COND_DIM, HIDDEN, LATENT, K = 3, 16, 256, 64, 1024
TCHUNK = 16
NPOS = 256
NVALID = N - 2

SC_NC, SC_NS = 2, 16
SC_NW = SC_NC * SC_NS
B_PER_W = B // SC_NW

NEG = float("-inf")


def _enc_kernel(x_ref, w9t_ref, cb_b_ref, w1_ref, b1_ref, w2_ref, b2_ref,
                wl_ref, bl_ref, cb_ref, ct_ref, nt_ref, wc_ref, wn_ref,
                bd_ref, idx_ref, part_ref):
    run = jnp.full((HIDDEN, B), NEG, jnp.float32)
    for ch in range(NPOS // TCHUNK):
        hp = jnp.dot(w9t_ref[:], x_ref[:, ch * TCHUNK * B:(ch + 1) * TCHUNK * B],
                     preferred_element_type=jnp.float32)
        if (ch + 1) * TCHUNK > NVALID:
            lane = lax.broadcasted_iota(jnp.int32, (HIDDEN, TCHUNK * B), 1)
            hp = jnp.where(lane < (NVALID - ch * TCHUNK) * B, hp, NEG)
        s = TCHUNK * B // 2
        while s >= B:
            hp = jnp.maximum(hp[:, :s], hp[:, s:2 * s])
            s //= 2
        run = jnp.maximum(run, hp)
    h = run + cb_b_ref[:]
    h = jnp.maximum(jnp.dot(w1_ref[:], h, preferred_element_type=jnp.float32)
                    + b1_ref[:], 0.0)
    h = jnp.maximum(jnp.dot(w2_ref[:], h, preferred_element_type=jnp.float32)
                    + b2_ref[:], 0.0)
    z = jnp.dot(wl_ref[:], h, preferred_element_type=jnp.float32) + bl_ref[:]
    cbn = jnp.sum(cb_ref[:] ** 2, axis=1, keepdims=True)
    dist = cbn - 2.0 * jnp.dot(cb_ref[:], z,
                               preferred_element_type=jnp.float32)
    dmin = jnp.min(dist, axis=0, keepdims=True)
    ks = lax.broadcasted_iota(jnp.int32, (K, B), 0)
    idx_ref[0, :] = jnp.min(jnp.where(dist <= dmin, ks, K), axis=0)
    part_ref[:, :] = (jnp.dot(wc_ref[:], ct_ref[:], preferred_element_type=jnp.float32)
                      + jnp.dot(wn_ref[:], nt_ref[:], preferred_element_type=jnp.float32)
                      + bd_ref[:])


def _dec_kernel(zq_ref, part_ref, wz_ref, w1_ref, b1_ref, w2_ref, b2_ref,
                wo_ref, bo_ref, out_ref):
    zt = zq_ref[:].T
    d = jnp.maximum(jnp.dot(wz_ref[:], zt,
                            preferred_element_type=jnp.float32) + part_ref[:], 0.0)
    d = jnp.maximum(jnp.dot(w1_ref[:], d, preferred_element_type=jnp.float32)
                    + b1_ref[:], 0.0)
    d = jnp.maximum(jnp.dot(w2_ref[:], d, preferred_element_type=jnp.float32)
                    + b2_ref[:], 0.0)
    out_ref[:, :] = (jnp.dot(wo_ref[:], d, preferred_element_type=jnp.float32)
                     + bo_ref[:]).T


def _sc_gather(codebook, idx):
    mesh = plsc.VectorSubcoreMesh(core_axis_name="c", subcore_axis_name="s")

    @functools.partial(
        pl.kernel, mesh=mesh,
        out_type=jax.ShapeDtypeStruct((B, LATENT), jnp.float32),
        compiler_params=pltpu.CompilerParams(use_tc_tiling_on_sc=False),
        scratch_types=[
            pltpu.VMEM((B_PER_W,), jnp.int32),
            pltpu.VMEM((B_PER_W, LATENT), jnp.float32),
            pltpu.SemaphoreType.DMA,
        ],
    )
    def gather(table_hbm, idx_hbm, out_hbm, idx_v, rows_v, sem):
        wid = lax.axis_index("s") * SC_NC + lax.axis_index("c")
        base = wid * B_PER_W
        pltpu.sync_copy(idx_hbm.at[pl.ds(base, B_PER_W)], idx_v)
        pltpu.async_copy(table_hbm.at[idx_v], rows_v, sem).wait()
        pltpu.sync_copy(rows_v, out_hbm.at[pl.ds(base, B_PER_W)])

    return gather(codebook, idx)


def kernel(x, c, noise, conv_w, conv_b, enc_h1_w, enc_h1_b, enc_h2_w, enc_h2_b,
           enc_lat_w, enc_lat_b, codebook, dec_in_w, dec_in_b, dec_h1_w, dec_h1_b,
           dec_h2_w, dec_h2_b, dec_out_w, dec_out_b):
    f32 = jnp.float32
    xp = jnp.pad(x, ((0, 0), (0, 2), (0, 0)))
    xt = jnp.transpose(xp, (2, 1, 0))
    x9 = jnp.stack([xt[cc, k:k + NPOS, :]
                    for k in range(3) for cc in range(3)])
    x9 = x9.reshape(9, NPOS * B).astype(jnp.bfloat16)
    w9t = jnp.transpose(conv_w, (2, 1, 0)).reshape(9, HIDDEN).T.astype(jnp.bfloat16)
    col = lambda v: v[:, None].astype(f32)
    wz = dec_in_w[:LATENT].T
    wc = dec_in_w[LATENT:LATENT + COND_DIM].T
    wn = dec_in_w[LATENT + COND_DIM:].T

    idx2, part = pl.pallas_call(
        _enc_kernel,
        out_shape=[
            jax.ShapeDtypeStruct((1, B), jnp.int32),
            jax.ShapeDtypeStruct((HIDDEN, B), f32),
        ],
    )(x9, w9t, col(conv_b), enc_h1_w.T, col(enc_h1_b), enc_h2_w.T,
      col(enc_h2_b), enc_lat_w.T, col(enc_lat_b), codebook, c.T, noise.T,
      wc, wn, col(dec_in_b))

    z_q = _sc_gather(codebook, idx2.reshape(B))

    out = pl.pallas_call(
        _dec_kernel,
        out_shape=jax.ShapeDtypeStruct((B, INPUT_DIM), f32),
    )(z_q, part, wz, dec_h1_w.T, col(dec_h1_b), dec_h2_w.T, col(dec_h2_b),
      dec_out_w.T, col(dec_out_b))
    return out


# speedup vs baseline: 2.2781x; 1.4479x over previous
import jax
import jax.numpy as jnp
from jax import lax
from jax.experimental import pallas as pl
from jax.experimental.pallas import tpu as pltpu

B, N = 512, 256
INPUT_DIM, COND_DIM, HIDDEN, LATENT, K = 3, 16, 256, 64, 1024
TCHUNK = 16
NPOS = 256
NVALID = N - 2
NEG = float("-inf")


def _fused_kernel(x_ref, w9t_ref, cb_b_ref, w1_ref, b1_ref, w2_ref, b2_ref,
                  wl_ref, bl_ref, cb_ref, cbt_ref, ct_ref, nt_ref, wc_ref,
                  wn_ref, bd_ref, dz_ref, d1_ref, db1_ref, d2_ref, db2_ref,
                  do_ref, dbo_ref, out_ref):
    run = jnp.full((HIDDEN, B), NEG, jnp.float32)
    for ch in range(NPOS // TCHUNK):
        hp = jnp.dot(w9t_ref[:], x_ref[:, ch * TCHUNK * B:(ch + 1) * TCHUNK * B],
                     preferred_element_type=jnp.float32)
        if (ch + 1) * TCHUNK > NVALID:
            lane = lax.broadcasted_iota(jnp.int32, (HIDDEN, TCHUNK * B), 1)
            hp = jnp.where(lane < (NVALID - ch * TCHUNK) * B, hp, NEG)
        s = TCHUNK * B // 2
        while s >= B:
            hp = jnp.maximum(hp[:, :s], hp[:, s:2 * s])
            s //= 2
        run = jnp.maximum(run, hp)
    h = run + cb_b_ref[:]
    h = jnp.maximum(jnp.dot(w1_ref[:], h, preferred_element_type=jnp.float32)
                    + b1_ref[:], 0.0)
    h = jnp.maximum(jnp.dot(w2_ref[:], h, preferred_element_type=jnp.float32)
                    + b2_ref[:], 0.0)
    z = jnp.dot(wl_ref[:], h, preferred_element_type=jnp.float32) + bl_ref[:]
    cbn = jnp.sum(cb_ref[:] ** 2, axis=1, keepdims=True)
    dist = cbn - 2.0 * jnp.dot(cb_ref[:], z, preferred_element_type=jnp.float32)
    dmin = jnp.min(dist, axis=0, keepdims=True)
    ks = lax.broadcasted_iota(jnp.int32, (K, B), 0)
    idx = jnp.min(jnp.where(dist <= dmin, ks, K), axis=0)
    oh = jnp.where(ks == idx[None, :], 1.0, 0.0).astype(jnp.float32)
    zq = jnp.dot(cbt_ref[:], oh, preferred_element_type=jnp.float32)
    part = (jnp.dot(wc_ref[:], ct_ref[:], preferred_element_type=jnp.float32)
            + jnp.dot(wn_ref[:], nt_ref[:], preferred_element_type=jnp.float32)
            + bd_ref[:])
    d = jnp.maximum(jnp.dot(dz_ref[:], zq, preferred_element_type=jnp.float32)
                    + part, 0.0)
    d = jnp.maximum(jnp.dot(d1_ref[:], d, preferred_element_type=jnp.float32)
                    + db1_ref[:], 0.0)
    d = jnp.maximum(jnp.dot(d2_ref[:], d, preferred_element_type=jnp.float32)
                    + db2_ref[:], 0.0)
    out_ref[:, :] = (jnp.dot(do_ref[:], d, preferred_element_type=jnp.float32)
                     + dbo_ref[:]).T


def kernel(x, c, noise, conv_w, conv_b, enc_h1_w, enc_h1_b, enc_h2_w, enc_h2_b,
           enc_lat_w, enc_lat_b, codebook, dec_in_w, dec_in_b, dec_h1_w, dec_h1_b,
           dec_h2_w, dec_h2_b, dec_out_w, dec_out_b):
    f32 = jnp.float32
    xp = jnp.pad(x, ((0, 0), (0, 2), (0, 0)))
    xt = jnp.transpose(xp, (2, 1, 0))
    x9 = jnp.stack([xt[cc, k:k + NPOS, :]
                    for k in range(3) for cc in range(3)])
    x9 = x9.reshape(9, NPOS * B).astype(jnp.bfloat16)
    w9t = jnp.transpose(conv_w, (2, 1, 0)).reshape(9, HIDDEN).T.astype(jnp.bfloat16)
    col = lambda v: v[:, None].astype(f32)
    wz = dec_in_w[:LATENT].T
    wc = dec_in_w[LATENT:LATENT + COND_DIM].T
    wn = dec_in_w[LATENT + COND_DIM:].T

    out = pl.pallas_call(
        _fused_kernel,
        out_shape=jax.ShapeDtypeStruct((B, INPUT_DIM), f32),
    )(x9, w9t, col(conv_b), enc_h1_w.T, col(enc_h1_b), enc_h2_w.T,
      col(enc_h2_b), enc_lat_w.T, col(enc_lat_b), codebook, codebook.T,
      c.T, noise.T, wc, wn, col(dec_in_b), wz, dec_h1_w.T, col(dec_h1_b),
      dec_h2_w.T, col(dec_h2_b), dec_out_w.T, col(dec_out_b))
    return out


# speedup vs baseline: 2.6414x; 1.1594x over previous
import jax
import jax.numpy as jnp
from jax import lax
from jax.experimental import pallas as pl
from jax.experimental.pallas import tpu as pltpu

B, N = 512, 256
INPUT_DIM, COND_DIM, HIDDEN, LATENT, K = 3, 16, 256, 64, 1024
TCHUNK = 16
NPOS = 256
NVALID = N - 2
NEG = float("-inf")

O_W1, O_W2, O_D1, O_D2, O_WZ, O_WC, O_WN, O_BIAS = (
    0, 256, 512, 768, 1024, 1088, 1104, 1152)
PK1_W = 1160
O_WL, O_BL, O_CBT = 0, 256, 384
PK2_W = 384 + K


def _fused_kernel(x_ref, w9t_ref, pk1_ref, pk2_ref, cb_ref, cn_ref, pk4_ref,
                  out_ref):
    f32 = jnp.float32
    run = jnp.full((HIDDEN, B), NEG, f32)
    for ch in range(NPOS // TCHUNK):
        hp = jnp.dot(w9t_ref[:], x_ref[:, ch * TCHUNK * B:(ch + 1) * TCHUNK * B],
                     preferred_element_type=f32)
        if (ch + 1) * TCHUNK > NVALID:
            lane = lax.broadcasted_iota(jnp.int32, (HIDDEN, TCHUNK * B), 1)
            hp = jnp.where(lane < (NVALID - ch * TCHUNK) * B, hp, NEG)
        s = TCHUNK * B // 2
        while s >= B:
            hp = jnp.maximum(hp[:, :s], hp[:, s:2 * s])
            s //= 2
        run = jnp.maximum(run, hp)
    bias = lambda i: pk1_ref[:, O_BIAS + i:O_BIAS + i + 1]
    h = run + bias(5)
    h = jnp.maximum(jnp.dot(pk1_ref[:, O_W1:O_W1 + HIDDEN], h,
                            preferred_element_type=f32) + bias(0), 0.0)
    h = jnp.maximum(jnp.dot(pk1_ref[:, O_W2:O_W2 + HIDDEN], h,
                            preferred_element_type=f32) + bias(1), 0.0)
    z = (jnp.dot(pk2_ref[:, O_WL:O_WL + HIDDEN], h, preferred_element_type=f32)
         + pk2_ref[:, O_BL:O_BL + 1])
    cbn = jnp.sum(cb_ref[:] ** 2, axis=1, keepdims=True)
    dist = cbn - 2.0 * jnp.dot(cb_ref[:], z, preferred_element_type=f32)
    dmin = jnp.min(dist, axis=0, keepdims=True)
    ks = lax.broadcasted_iota(jnp.int32, (K, B), 0)
    idx = jnp.min(jnp.where(dist <= dmin, ks, K), axis=0)
    oh = jnp.where(ks == idx[None, :], 1.0, 0.0).astype(f32)
    zq = jnp.dot(pk2_ref[:, O_CBT:O_CBT + K], oh,
                 preferred_element_type=f32)
    part = (jnp.dot(pk1_ref[:, O_WC:O_WC + COND_DIM], cn_ref[:COND_DIM, :],
                    preferred_element_type=f32)
            + jnp.dot(pk1_ref[:, O_WN:O_WN + INPUT_DIM], cn_ref[COND_DIM:, :],
                      preferred_element_type=f32) + bias(2))
    d = jnp.maximum(jnp.dot(pk1_ref[:, O_WZ:O_WZ + LATENT], zq,
                            preferred_element_type=f32) + part, 0.0)
    d = jnp.maximum(jnp.dot(pk1_ref[:, O_D1:O_D1 + HIDDEN], d,
                            preferred_element_type=f32) + bias(3), 0.0)
    d = jnp.maximum(jnp.dot(pk1_ref[:, O_D2:O_D2 + HIDDEN], d,
                            preferred_element_type=f32) + bias(4), 0.0)
    out_ref[:, :] = (jnp.dot(pk4_ref[:, :HIDDEN], d, preferred_element_type=f32)
                     + pk4_ref[:, HIDDEN:HIDDEN + 1]).T


def kernel(x, c, noise, conv_w, conv_b, enc_h1_w, enc_h1_b, enc_h2_w, enc_h2_b,
           enc_lat_w, enc_lat_b, codebook, dec_in_w, dec_in_b, dec_h1_w, dec_h1_b,
           dec_h2_w, dec_h2_b, dec_out_w, dec_out_b):
    f32 = jnp.float32
    xp = jnp.pad(x, ((0, 0), (0, 2), (0, 0)))
    xt = jnp.transpose(xp, (2, 1, 0))
    x9 = jnp.stack([xt[cc, k:k + NPOS, :]
                    for k in range(3) for cc in range(3)])
    x9 = x9.reshape(9, NPOS * B).astype(jnp.bfloat16)
    w9t = jnp.transpose(conv_w, (2, 1, 0)).reshape(9, HIDDEN).T.astype(jnp.bfloat16)
    col = lambda v: v[:, None].astype(f32)
    zpad = lambda w, n: jnp.concatenate(
        [w, jnp.zeros((HIDDEN, n - w.shape[1]), f32)], axis=1) if n > w.shape[1] else w
    pk1 = jnp.concatenate([
        enc_h1_w.T, enc_h2_w.T, dec_h1_w.T, dec_h2_w.T,
        dec_in_w[:LATENT].T,
        dec_in_w[LATENT:LATENT + COND_DIM].T,
        zpad(dec_in_w[LATENT + COND_DIM:].T, O_BIAS - O_WN),
        col(enc_h1_b), col(enc_h2_b), col(dec_in_b), col(dec_h1_b),
        col(dec_h2_b), col(conv_b),
        jnp.zeros((HIDDEN, PK1_W - O_BIAS - 6), f32)], axis=1)
    pk2 = jnp.concatenate([
        enc_lat_w.T, enc_lat_b[:, None].astype(f32),
        jnp.zeros((LATENT, O_CBT - O_BL - 1), f32), codebook.T], axis=1)
    cn = jnp.concatenate([c.T, noise.T], axis=0)
    pk4 = jnp.concatenate([dec_out_w.T, dec_out_b[:, None].astype(f32)], axis=1)

    out = pl.pallas_call(
        _fused_kernel,
        out_shape=jax.ShapeDtypeStruct((B, INPUT_DIM), f32),
    )(x9, w9t, pk1, pk2, codebook, cn, pk4)
    return out
